# baseline (speedup 1.0000x reference)
import functools

import jax
import jax.numpy as jnp
from jax import lax
from jax.experimental import pallas as pl
from jax.experimental.pallas import tpu as pltpu
from jax.experimental.pallas import tpu_sc as plsc

NC = 2
NS = 16
NW = NC * NS
GRP = 128


def _tables_body(src_ref, dst_ref, wst_ref, wdt_ref, b_ref, ts_ref, td_ref):
    ts_ref[...] = jnp.dot(src_ref[...], wst_ref[...],
                          preferred_element_type=jnp.float32)
    td_ref[...] = jnp.dot(dst_ref[...], wdt_ref[...],
                          preferred_element_type=jnp.float32) + b_ref[...]


def _edge_body(ef_ref, g_ref, wet_ref, wot_ref, bo_ref, gam_ref, bet_ref,
               out_ref):
    s = jnp.dot(ef_ref[...], wet_ref[...],
                preferred_element_type=jnp.float32) + g_ref[...]
    h = s * jax.nn.sigmoid(s)
    o = jnp.dot(h, wot_ref[...], preferred_element_type=jnp.float32)
    o = o + bo_ref[...]
    mu = jnp.mean(o, axis=-1, keepdims=True)
    var = jnp.mean((o - mu) ** 2, axis=-1, keepdims=True)
    out_ref[...] = ((o - mu) * lax.rsqrt(var + 1e-5)) * gam_ref[...] + bet_ref[...]


def _sc_gather_sum(R_pad, H, steps):
    mesh = plsc.VectorSubcoreMesh(core_axis_name="c", subcore_axis_name="s")
    assert steps % 2 == 0

    @functools.partial(
        pl.kernel,
        mesh=mesh,
        out_type=jax.ShapeDtypeStruct((R_pad, GRP, H), jnp.float32),
        scratch_types=[
            pltpu.VMEM((steps, GRP), jnp.int32),
            pltpu.VMEM((steps, GRP), jnp.int32),
            pltpu.VMEM((2, GRP, H), jnp.float32),
            pltpu.VMEM((2, GRP, H), jnp.float32),
            pltpu.SemaphoreType.DMA,
            pltpu.SemaphoreType.DMA,
            pltpu.SemaphoreType.DMA,
            pltpu.SemaphoreType.DMA,
            pltpu.SemaphoreType.DMA,
            pltpu.SemaphoreType.DMA,
        ],
    )
    def gather_sum(ts_hbm, td_hbm, si_hbm, di_hbm, g_hbm,
                   si_v, di_v, rs_v, rd_v,
                   gs0, gd0, gs1, gd1, w0, w1):
        wid = lax.axis_index("c") * NS + lax.axis_index("s")
        base = wid * steps
        gsem = (gs0, gs1)
        dsem = (gd0, gd1)
        wsem = (w0, w1)

        pltpu.sync_copy(si_hbm.at[pl.ds(base, steps)], si_v)
        pltpu.sync_copy(di_hbm.at[pl.ds(base, steps)], di_v)

        def issue_gather(t, p):
            pltpu.async_copy(ts_hbm.at[si_v.at[t]], rs_v.at[p], gsem[p])
            pltpu.async_copy(td_hbm.at[di_v.at[t]], rd_v.at[p], dsem[p])

        def wait_gather(p):
            pltpu.make_async_copy(ts_hbm.at[si_v.at[0]], rs_v.at[p],
                                  gsem[p]).wait()
            pltpu.make_async_copy(td_hbm.at[di_v.at[0]], rd_v.at[p],
                                  dsem[p]).wait()

        def wait_write(p):
            pltpu.make_async_copy(rs_v.at[p], g_hbm.at[base], wsem[p]).wait()

        def add_and_write(t, p):
            @pl.loop(0, GRP)
            def _(i):
                for j in range(H // 16):
                    sl = (p, i, pl.ds(j * 16, 16))
                    rs_v[sl] = rs_v[sl] + rd_v[sl]

            pltpu.async_copy(rs_v.at[p], g_hbm.at[base + t], wsem[p])

        issue_gather(0, 0)

        @pl.loop(0, steps, step=2)
        def _(t):
            @pl.when(t >= 2)
            def _():
                wait_write(1)

            issue_gather(t + 1, 1)
            wait_gather(0)
            add_and_write(t, 0)

            @pl.when(t + 2 < steps)
            def _():
                wait_write(0)
                issue_gather(t + 2, 0)

            wait_gather(1)
            add_and_write(t + 1, 1)

        wait_write(0)
        wait_write(1)

    return gather_sum


def kernel(efeat, src_feat, dst_feat, src_idx, dst_idx, W_e, W_s, W_d, b,
           W_out, b_out, gamma, beta):
    E, EF = efeat.shape
    N, D = src_feat.shape
    H = W_s.shape[0]
    OUT = W_out.shape[0]

    NB = 2000
    tables = pl.pallas_call(
        _tables_body,
        grid=(N // NB,),
        in_specs=[
            pl.BlockSpec((NB, D), lambda i: (i, 0)),
            pl.BlockSpec((NB, D), lambda i: (i, 0)),
            pl.BlockSpec((D, H), lambda i: (0, 0)),
            pl.BlockSpec((D, H), lambda i: (0, 0)),
            pl.BlockSpec((1, H), lambda i: (0, 0)),
        ],
        out_specs=[
            pl.BlockSpec((NB, H), lambda i: (i, 0)),
            pl.BlockSpec((NB, H), lambda i: (i, 0)),
        ],
        out_shape=[
            jax.ShapeDtypeStruct((N, H), jnp.float32),
            jax.ShapeDtypeStruct((N, H), jnp.float32),
        ],
    )
    T_s, T_d = tables(src_feat, dst_feat, W_s.T, W_d.T, b.reshape(1, H))

    R = E // GRP
    steps = 2 * ((R + 2 * NW - 1) // (2 * NW))
    R_pad = NW * steps
    si = src_idx.astype(jnp.int32)
    di = dst_idx.astype(jnp.int32)
    if R_pad > R:
        pad = jnp.zeros(((R_pad - R) * GRP,), jnp.int32)
        si = jnp.concatenate([si, pad])
        di = jnp.concatenate([di, pad])
    si = si.reshape(R_pad, GRP)
    di = di.reshape(R_pad, GRP)
    g = _sc_gather_sum(R_pad, H, steps)(T_s, T_d, si, di)
    g = g.reshape(R_pad * GRP, H)

    BE = 3200
    out = pl.pallas_call(
        _edge_body,
        grid=(E // BE,),
        in_specs=[
            pl.BlockSpec((BE, EF), lambda i: (i, 0)),
            pl.BlockSpec((BE, H), lambda i: (i, 0)),
            pl.BlockSpec((EF, H), lambda i: (0, 0)),
            pl.BlockSpec((H, OUT), lambda i: (0, 0)),
            pl.BlockSpec((1, OUT), lambda i: (0, 0)),
            pl.BlockSpec((1, OUT), lambda i: (0, 0)),
            pl.BlockSpec((1, OUT), lambda i: (0, 0)),
        ],
        out_specs=pl.BlockSpec((BE, OUT), lambda i: (i, 0)),
        out_shape=jax.ShapeDtypeStruct((E, OUT), jnp.float32),
    )(efeat, g, W_e.T, W_out.T, b_out.reshape(1, OUT),
      gamma.reshape(1, OUT), beta.reshape(1, OUT))
    return out


# speedup vs baseline: 1.7909x; 1.7909x over previous
import functools

import jax
import jax.numpy as jnp
from jax import lax
from jax.experimental import pallas as pl
from jax.experimental.pallas import tpu as pltpu
from jax.experimental.pallas import tpu_sc as plsc

NC = 2
NS = 16
NW = NC * NS
GRP = 128


def _tables_body(src_ref, dst_ref, wst_ref, wdt_ref, b_ref, ts_ref, td_ref):
    ts_ref[...] = jnp.dot(src_ref[...], wst_ref[...],
                          preferred_element_type=jnp.float32)
    td_ref[...] = jnp.dot(dst_ref[...], wdt_ref[...],
                          preferred_element_type=jnp.float32) + b_ref[...]


def _edge_body(ef_ref, g_ref, wet_ref, wot_ref, bo_ref, gam_ref, bet_ref,
               out_ref):
    s = jnp.dot(ef_ref[...], wet_ref[...],
                preferred_element_type=jnp.float32) + g_ref[...]
    h = s * jax.nn.sigmoid(s)
    o = jnp.dot(h, wot_ref[...], preferred_element_type=jnp.float32)
    o = o + bo_ref[...]
    mu = jnp.mean(o, axis=-1, keepdims=True)
    var = jnp.mean((o - mu) ** 2, axis=-1, keepdims=True)
    out_ref[...] = ((o - mu) * lax.rsqrt(var + 1e-5)) * gam_ref[...] + bet_ref[...]


def _sc_gather_sum(R_pad, H, steps):
    mesh = plsc.VectorSubcoreMesh(core_axis_name="c", subcore_axis_name="s")
    assert steps % 2 == 0

    @functools.partial(
        pl.kernel,
        mesh=mesh,
        out_type=jax.ShapeDtypeStruct((R_pad, GRP, H), jnp.float32),
        scratch_types=[
            pltpu.VMEM((steps, GRP), jnp.int32),
            pltpu.VMEM((steps, GRP), jnp.int32),
            pltpu.VMEM((2, GRP, H), jnp.float32),
            pltpu.VMEM((2, GRP, H), jnp.float32),
            pltpu.SemaphoreType.DMA,
            pltpu.SemaphoreType.DMA,
            pltpu.SemaphoreType.DMA,
            pltpu.SemaphoreType.DMA,
            pltpu.SemaphoreType.DMA,
            pltpu.SemaphoreType.DMA,
        ],
    )
    def gather_sum(ts_hbm, td_hbm, si_hbm, di_hbm, g_hbm,
                   si_v, di_v, rs_v, rd_v,
                   gs0, gd0, gs1, gd1, w0, w1):
        wid = lax.axis_index("c") * NS + lax.axis_index("s")
        base = wid * steps
        gsem = (gs0, gs1)
        dsem = (gd0, gd1)
        wsem = (w0, w1)

        pltpu.sync_copy(si_hbm.at[pl.ds(base, steps)], si_v)
        pltpu.sync_copy(di_hbm.at[pl.ds(base, steps)], di_v)

        def issue_gather(t, p):
            pltpu.async_copy(ts_hbm.at[si_v.at[t]], rs_v.at[p], gsem[p])
            pltpu.async_copy(td_hbm.at[di_v.at[t]], rd_v.at[p], dsem[p])

        def wait_gather(p):
            pltpu.make_async_copy(ts_hbm.at[si_v.at[0]], rs_v.at[p],
                                  gsem[p]).wait()
            pltpu.make_async_copy(td_hbm.at[di_v.at[0]], rd_v.at[p],
                                  dsem[p]).wait()

        def wait_write(p):
            pltpu.make_async_copy(rs_v.at[p], g_hbm.at[base], wsem[p]).wait()

        def add_and_write(t, p):
            @pl.loop(0, GRP)
            def _(i):
                for j in range(H // 16):
                    sl = (p, i, pl.ds(j * 16, 16))
                    rs_v[sl] = rs_v[sl] + rd_v[sl]

            pltpu.async_copy(rs_v.at[p], g_hbm.at[base + t], wsem[p])

        issue_gather(0, 0)

        @pl.loop(0, steps, step=2)
        def _(t):
            @pl.when(t >= 2)
            def _():
                wait_write(1)

            issue_gather(t + 1, 1)
            wait_gather(0)
            add_and_write(t, 0)

            @pl.when(t + 2 < steps)
            def _():
                wait_write(0)
                issue_gather(t + 2, 0)

            wait_gather(1)
            add_and_write(t + 1, 1)

        wait_write(0)
        wait_write(1)

    return gather_sum


def kernel(efeat, src_feat, dst_feat, src_idx, dst_idx, W_e, W_s, W_d, b,
           W_out, b_out, gamma, beta):
    E, EF = efeat.shape
    N, D = src_feat.shape
    H = W_s.shape[0]
    OUT = W_out.shape[0]

    NB = 2000
    tables = pl.pallas_call(
        _tables_body,
        grid=(N // NB,),
        in_specs=[
            pl.BlockSpec((NB, D), lambda i: (i, 0)),
            pl.BlockSpec((NB, D), lambda i: (i, 0)),
            pl.BlockSpec((D, H), lambda i: (0, 0)),
            pl.BlockSpec((D, H), lambda i: (0, 0)),
            pl.BlockSpec((1, H), lambda i: (0, 0)),
        ],
        out_specs=[
            pl.BlockSpec((NB, H), lambda i: (i, 0)),
            pl.BlockSpec((NB, H), lambda i: (i, 0)),
        ],
        out_shape=[
            jax.ShapeDtypeStruct((N, H), jnp.float32),
            jax.ShapeDtypeStruct((N, H), jnp.float32),
        ],
    )
    T_s, T_d = tables(src_feat, dst_feat, W_s.T, W_d.T, b.reshape(1, H))

    R = E // GRP
    steps = 2 * ((R + 2 * NW - 1) // (2 * NW))
    R_pad = NW * steps
    si = src_idx.astype(jnp.int32)
    di = dst_idx.astype(jnp.int32)
    if R_pad > R:
        pad = jnp.arange((R_pad - R) * GRP, dtype=jnp.int32) % N
        si = jnp.concatenate([si, pad])
        di = jnp.concatenate([di, pad])
    si = si.reshape(R_pad, GRP)
    di = di.reshape(R_pad, GRP)
    g = _sc_gather_sum(R_pad, H, steps)(T_s, T_d, si, di)
    g = g.reshape(R_pad * GRP, H)

    BE = 3200
    out = pl.pallas_call(
        _edge_body,
        grid=(E // BE,),
        in_specs=[
            pl.BlockSpec((BE, EF), lambda i: (i, 0)),
            pl.BlockSpec((BE, H), lambda i: (i, 0)),
            pl.BlockSpec((EF, H), lambda i: (0, 0)),
            pl.BlockSpec((H, OUT), lambda i: (0, 0)),
            pl.BlockSpec((1, OUT), lambda i: (0, 0)),
            pl.BlockSpec((1, OUT), lambda i: (0, 0)),
            pl.BlockSpec((1, OUT), lambda i: (0, 0)),
        ],
        out_specs=pl.BlockSpec((BE, OUT), lambda i: (i, 0)),
        out_shape=jax.ShapeDtypeStruct((E, OUT), jnp.float32),
    )(efeat, g, W_e.T, W_out.T, b_out.reshape(1, OUT),
      gamma.reshape(1, OUT), beta.reshape(1, OUT))
    return out


# speedup vs baseline: 1.8755x; 1.0473x over previous
import functools

import jax
import jax.numpy as jnp
from jax import lax
from jax.experimental import pallas as pl
from jax.experimental.pallas import tpu as pltpu
from jax.experimental.pallas import tpu_sc as plsc

NC = 2
NS = 16
NW = NC * NS
GRP = 128


def _tables_body(src_ref, dst_ref, wst_ref, wdt_ref, b_ref, ts_ref, td_ref):
    ts_ref[...] = jnp.dot(src_ref[...], wst_ref[...],
                          preferred_element_type=jnp.float32)
    td_ref[...] = jnp.dot(dst_ref[...], wdt_ref[...],
                          preferred_element_type=jnp.float32) + b_ref[...]


def _edge_body(ef_ref, g_ref, wet_ref, wot_ref, bo_ref, gam_ref, bet_ref,
               out_ref):
    s = jnp.dot(ef_ref[...], wet_ref[...],
                preferred_element_type=jnp.float32) + g_ref[...]
    h = s * jax.nn.sigmoid(s)
    o = jnp.dot(h, wot_ref[...], preferred_element_type=jnp.float32)
    o = o + bo_ref[...]
    mu = jnp.mean(o, axis=-1, keepdims=True)
    var = jnp.mean((o - mu) ** 2, axis=-1, keepdims=True)
    out_ref[...] = ((o - mu) * lax.rsqrt(var + 1e-5)) * gam_ref[...] + bet_ref[...]


def _sc_gather_sum(R_pad, H, steps):
    mesh = plsc.VectorSubcoreMesh(core_axis_name="c", subcore_axis_name="s")
    assert steps % 2 == 0

    @functools.partial(
        pl.kernel,
        mesh=mesh,
        out_type=jax.ShapeDtypeStruct((R_pad, GRP, H), jnp.float32),
        scratch_types=[
            pltpu.VMEM((steps, GRP), jnp.int32),
            pltpu.VMEM((steps, GRP), jnp.int32),
            pltpu.VMEM((2, GRP, H), jnp.float32),
            pltpu.VMEM((2, GRP, H), jnp.float32),
            pltpu.SemaphoreType.DMA,
            pltpu.SemaphoreType.DMA,
            pltpu.SemaphoreType.DMA,
            pltpu.SemaphoreType.DMA,
            pltpu.SemaphoreType.DMA,
            pltpu.SemaphoreType.DMA,
        ],
    )
    def gather_sum(ts_hbm, td_hbm, si_hbm, di_hbm, g_hbm,
                   si_v, di_v, rs_v, rd_v,
                   gs0, gd0, gs1, gd1, w0, w1):
        wid = lax.axis_index("c") * NS + lax.axis_index("s")
        base = wid * steps
        gsem = (gs0, gs1)
        dsem = (gd0, gd1)
        wsem = (w0, w1)

        pltpu.sync_copy(si_hbm.at[pl.ds(base, steps)], si_v)
        pltpu.sync_copy(di_hbm.at[pl.ds(base, steps)], di_v)

        def issue_gather(t, p):
            pltpu.async_copy(ts_hbm.at[si_v.at[t]], rs_v.at[p], gsem[p])
            pltpu.async_copy(td_hbm.at[di_v.at[t]], rd_v.at[p], dsem[p])

        def wait_gather(p):
            pltpu.make_async_copy(ts_hbm.at[si_v.at[0]], rs_v.at[p],
                                  gsem[p]).wait()
            pltpu.make_async_copy(td_hbm.at[di_v.at[0]], rd_v.at[p],
                                  dsem[p]).wait()

        def wait_write(p):
            pltpu.make_async_copy(rs_v.at[p], g_hbm.at[base], wsem[p]).wait()

        def add_and_write(t, p):
            @pl.loop(0, GRP)
            def _(i):
                for j in range(H // 16):
                    sl = (p, i, pl.ds(j * 16, 16))
                    rs_v[sl] = rs_v[sl] + rd_v[sl]

            pltpu.async_copy(rs_v.at[p], g_hbm.at[base + t], wsem[p])

        issue_gather(0, 0)

        @pl.loop(0, steps, step=2)
        def _(t):
            @pl.when(t >= 2)
            def _():
                wait_write(1)

            issue_gather(t + 1, 1)
            wait_gather(0)
            add_and_write(t, 0)

            @pl.when(t + 2 < steps)
            def _():
                wait_write(0)
                issue_gather(t + 2, 0)

            wait_gather(1)
            add_and_write(t + 1, 1)

        wait_write(0)
        wait_write(1)

    return gather_sum


def kernel(efeat, src_feat, dst_feat, src_idx, dst_idx, W_e, W_s, W_d, b,
           W_out, b_out, gamma, beta):
    E, EF = efeat.shape
    N, D = src_feat.shape
    H = W_s.shape[0]
    OUT = W_out.shape[0]

    NB = 2000
    tables = pl.pallas_call(
        _tables_body,
        grid=(N // NB,),
        in_specs=[
            pl.BlockSpec((NB, D), lambda i: (i, 0)),
            pl.BlockSpec((NB, D), lambda i: (i, 0)),
            pl.BlockSpec((D, H), lambda i: (0, 0)),
            pl.BlockSpec((D, H), lambda i: (0, 0)),
            pl.BlockSpec((1, H), lambda i: (0, 0)),
        ],
        out_specs=[
            pl.BlockSpec((NB, H), lambda i: (i, 0)),
            pl.BlockSpec((NB, H), lambda i: (i, 0)),
        ],
        out_shape=[
            jax.ShapeDtypeStruct((N, H), jnp.float32),
            jax.ShapeDtypeStruct((N, H), jnp.float32),
        ],
    )
    T_s, T_d = tables(src_feat, dst_feat, W_s.T, W_d.T, b.reshape(1, H))

    K = 5
    EC = E // K
    RC = EC // GRP
    steps = 8 * ((RC + 8 * NW - 1) // (8 * NW))
    RCP = NW * steps
    si = src_idx.astype(jnp.int32)
    di = dst_idx.astype(jnp.int32)
    pad = jnp.arange((RCP - RC) * GRP, dtype=jnp.int32) % N

    BE = 3200
    BPC = EC // BE
    sc_call = _sc_gather_sum(RCP, H, steps)
    edge_in_specs = [
        pl.BlockSpec((BE, EF), lambda i: (i, 0)),
        pl.BlockSpec((BE, H), lambda i: (i, 0)),
        pl.BlockSpec((EF, H), lambda i: (0, 0)),
        pl.BlockSpec((H, OUT), lambda i: (0, 0)),
        pl.BlockSpec((1, OUT), lambda i: (0, 0)),
        pl.BlockSpec((1, OUT), lambda i: (0, 0)),
        pl.BlockSpec((1, OUT), lambda i: (0, 0)),
    ]
    consts = (W_e.T, W_out.T, b_out.reshape(1, OUT),
              gamma.reshape(1, OUT), beta.reshape(1, OUT))

    gs = []
    for k in range(K):
        si_k = jnp.concatenate([si[k * EC:(k + 1) * EC], pad]).reshape(RCP, GRP)
        di_k = jnp.concatenate([di[k * EC:(k + 1) * EC], pad]).reshape(RCP, GRP)
        gs.append(sc_call(T_s, T_d, si_k, di_k).reshape(RCP * GRP, H))

    out = None
    for k in range(K):
        off = k * BPC

        def ef_map(i, off=off):
            return (i + off, 0)

        specs = list(edge_in_specs)
        specs[0] = pl.BlockSpec((BE, EF), ef_map)
        args = (efeat, gs[k]) + consts
        if k == 0:
            out = pl.pallas_call(
                _edge_body,
                grid=(BPC,),
                in_specs=specs,
                out_specs=pl.BlockSpec((BE, OUT), ef_map),
                out_shape=jax.ShapeDtypeStruct((E, OUT), jnp.float32),
            )(*args)
        else:
            def edge_alias_body(o_in_ref, *refs):
                _edge_body(*refs)

            out = pl.pallas_call(
                edge_alias_body,
                grid=(BPC,),
                in_specs=[pl.BlockSpec(memory_space=pl.ANY)] + specs,
                out_specs=pl.BlockSpec((BE, OUT), ef_map),
                out_shape=jax.ShapeDtypeStruct((E, OUT), jnp.float32),
                input_output_aliases={0: 0},
            )(out, *args)
    return out
